# baseline (speedup 1.0000x reference)
import functools

import jax
import jax.numpy as jnp
from jax import lax
from jax.experimental import pallas as pl
from jax.experimental.pallas import tpu as pltpu
from jax.experimental.pallas import tpu_sc as plsc

D_MODEL = 64
SCALE = float(D_MODEL) ** 0.5

_INFO = plsc.get_sparse_core_info()
_NC = _INFO.num_cores
_NS = _INFO.num_subcores
_NW = _NC * _NS
_LANES = _INFO.num_lanes

_IW = 128
_CHUNK = 512
_IDX_ROWS = _CHUNK // _IW


@functools.partial(jax.jit, static_argnames=("n_rows",))
def _embed(x2d, table, n_rows):
    n_idx_rows = x2d.shape[0]
    rows_per_w = n_rows // _NW
    idx_rows_per_w = n_idx_rows // _NW
    chunks = rows_per_w // _CHUNK

    mesh = plsc.VectorSubcoreMesh(core_axis_name="c", subcore_axis_name="s")

    @functools.partial(
        pl.kernel,
        mesh=mesh,
        out_type=jax.ShapeDtypeStruct((n_rows, D_MODEL), jnp.float32),
        scratch_types=[
            pltpu.VMEM((_IDX_ROWS, _IW), jnp.int32),
            pltpu.VMEM((_CHUNK, D_MODEL), jnp.float32),
            pltpu.SemaphoreType.DMA,
        ],
        compiler_params=pltpu.CompilerParams(use_tc_tiling_on_sc=False),
    )
    def k(x_hbm, table_hbm, out_hbm, idx_v, rows_v, gsem):
        wid = lax.axis_index("s") * _NC + lax.axis_index("c")
        idx_row0 = wid * idx_rows_per_w
        out_base = wid * rows_per_w

        def chunk_body(t, _):
            pltpu.sync_copy(
                x_hbm.at[pl.ds(idx_row0 + t * _IDX_ROWS, _IDX_ROWS)], idx_v
            )
            descs = []
            for j in range(_IDX_ROWS):
                descs.append(
                    pltpu.async_copy(
                        table_hbm.at[idx_v.at[j]],
                        rows_v.at[pl.ds(j * _IW, _IW)],
                        gsem,
                    )
                )
            for d in descs:
                d.wait()

            def scale_body(r, _):
                for c in range(D_MODEL // _LANES):
                    sl = pl.ds(c * _LANES, _LANES)
                    rows_v[r, sl] = rows_v[r, sl] * SCALE
                return ()

            lax.fori_loop(0, _CHUNK, scale_body, ())

            pltpu.sync_copy(
                rows_v, out_hbm.at[pl.ds(out_base + t * _CHUNK, _CHUNK)]
            )
            return ()

        lax.fori_loop(0, chunks, chunk_body, ())

    return k(x2d, table)


def kernel(x, table):
    b0, b1 = x.shape
    n_rows = b0 * b1
    x2d = x.reshape(n_rows // _IW, _IW).astype(jnp.int32)
    out = _embed(x2d, table, n_rows)
    return out.reshape(b0, b1, D_MODEL)


# speedup vs baseline: 1.0252x; 1.0252x over previous
import functools

import jax
import jax.numpy as jnp
from jax import lax
from jax.experimental import pallas as pl
from jax.experimental.pallas import tpu as pltpu
from jax.experimental.pallas import tpu_sc as plsc

D_MODEL = 64
SCALE = float(D_MODEL) ** 0.5

_INFO = plsc.get_sparse_core_info()
_NC = _INFO.num_cores
_NS = _INFO.num_subcores
_NW = _NC * _NS
_LANES = _INFO.num_lanes

_IW = 128
_IDX_ROWS = 8
_SUB = 2
_CHUNK = _SUB * _IW


@functools.partial(jax.jit, static_argnames=("n_rows",))
def _embed(x2d, table, n_rows):
    n_idx_rows = x2d.shape[0]
    rows_per_w = n_rows // _NW
    idx_rows_per_w = n_idx_rows // _NW
    chunks = idx_rows_per_w // _IDX_ROWS

    mesh = plsc.VectorSubcoreMesh(core_axis_name="c", subcore_axis_name="s")

    @functools.partial(
        pl.kernel,
        mesh=mesh,
        out_type=jax.ShapeDtypeStruct((n_rows // 2, 2 * D_MODEL), jnp.float32),
        scratch_types=[
            pltpu.VMEM((_IDX_ROWS, _IW), jnp.int32),
            pltpu.VMEM((_CHUNK, D_MODEL), jnp.float32),
            pltpu.VMEM((_CHUNK // 2, 2 * D_MODEL), jnp.float32),
            pltpu.SemaphoreType.DMA,
        ],
        compiler_params=pltpu.CompilerParams(use_tc_tiling_on_sc=False),
    )
    def k(x_hbm, table_hbm, out_hbm, idx_v, rows_v, pack_v, gsem):
        wid = lax.axis_index("s") * _NC + lax.axis_index("c")
        idx_row0 = wid * idx_rows_per_w
        pair_base = wid * (rows_per_w // 2)

        def chunk_body(t, _):
            pltpu.sync_copy(
                x_hbm.at[pl.ds(idx_row0 + t * _IDX_ROWS, _IDX_ROWS)], idx_v
            )
            for s in range(_IDX_ROWS // _SUB):
                descs = []
                for j in range(_SUB):
                    descs.append(
                        pltpu.async_copy(
                            table_hbm.at[idx_v.at[s * _SUB + j]],
                            rows_v.at[pl.ds(j * _IW, _IW)],
                            gsem,
                        )
                    )
                for d in descs:
                    d.wait()

                def pack_body(p, _):
                    for rr in range(2):
                        for c in range(D_MODEL // _LANES):
                            src = pl.ds(c * _LANES, _LANES)
                            dst = pl.ds(rr * D_MODEL + c * _LANES, _LANES)
                            pack_v[p, dst] = rows_v[2 * p + rr, src] * SCALE
                    return ()

                lax.fori_loop(0, _CHUNK // 2, pack_body, ())

                pltpu.sync_copy(
                    pack_v,
                    out_hbm.at[
                        pl.ds(pair_base
                              + (t * (_IDX_ROWS // _SUB) + s) * (_CHUNK // 2),
                              _CHUNK // 2)
                    ],
                )
            return ()

        lax.fori_loop(0, chunks, chunk_body, ())

    return k(x2d, table)


def kernel(x, table):
    b0, b1 = x.shape
    n_rows = b0 * b1
    x2d = x.reshape(n_rows // _IW, _IW).astype(jnp.int32)
    out = _embed(x2d, table, n_rows)
    return out.reshape(n_rows, D_MODEL).reshape(b0, b1, D_MODEL)
